# baseline (speedup 1.0000x reference)
import jax
import jax.numpy as jnp
from jax import lax
from jax.experimental import pallas as pl
from jax.experimental.pallas import tpu as pltpu
from jax.experimental.pallas import tpu_sc as plsc

NUM_NODES = 1000000
NUM_RELATIONS = 1000
HIDDEN = 64
BATCH = 16384

NC = 2
NS = 16
L = 16
NW = NC * NS
B_PER_W = BATCH // NW
IDX_CHUNK = 128
N_CHUNKS = B_PER_W // IDX_CHUNK


def _neg_sqrt(x):
    i = plsc.bitcast(x, jnp.int32)
    i = jnp.int32(0x5F3759DF) - lax.shift_right_arithmetic(i, jnp.int32(1))
    y = plsc.bitcast(i, jnp.float32)
    half_x = x * jnp.float32(0.5)
    for _ in range(3):
        y = y * (jnp.float32(1.5) - half_x * y * y)
    return -(x * y)


def _tec_body(head_hbm, rel_hbm, tail_hbm, node_hbm, relemb_hbm, out_hbm,
              idx_h, idx_r, idx_t, h_rows, r_rows, t_rows, ssq, sem):
    wid = lax.axis_index("s") * NC + lax.axis_index("c")
    base = wid * B_PER_W

    csl = pl.ds(wid * N_CHUNKS, N_CHUNKS)
    pltpu.sync_copy(head_hbm.at[csl], idx_h)
    pltpu.sync_copy(rel_hbm.at[csl], idx_r)
    pltpu.sync_copy(tail_hbm.at[csl], idx_t)

    copies = []
    for j in range(N_CHUNKS):
        sl = pl.ds(j * IDX_CHUNK, IDX_CHUNK)
        copies.append(pltpu.async_copy(node_hbm.at[idx_h.at[j]], h_rows.at[sl], sem))
        copies.append(pltpu.async_copy(relemb_hbm.at[idx_r.at[j]], r_rows.at[sl], sem))
        copies.append(pltpu.async_copy(node_hbm.at[idx_t.at[j]], t_rows.at[sl], sem))
    for c in copies:
        c.wait()

    last_lane = lax.iota(jnp.int32, L) == (L - 1)

    def row_step(r, _):
        acc = jnp.zeros((L,), jnp.float32)
        for j in range(HIDDEN // L):
            sl = pl.ds(j * L, L)
            d = h_rows[r, sl] + r_rows[r, sl] - t_rows[r, sl]
            acc = acc + d * d
        c = plsc.cumsum(acc)
        plsc.store_scatter(ssq, [jnp.full((L,), r, jnp.int32)], c, mask=last_lane)
        return 0

    lax.fori_loop(0, B_PER_W, row_step, 0, unroll=4)

    def sqrt_step(g, _):
        sl = pl.ds(g * L, L)
        ssq[sl] = _neg_sqrt(ssq[sl] + jnp.float32(1e-12))
        return 0

    lax.fori_loop(0, B_PER_W // L, sqrt_step, 0, unroll=4)

    pltpu.sync_copy(ssq, out_hbm.at[pl.ds(base, B_PER_W)])


@jax.jit
def _kge_score(head, rel, tail, node_emb, rel_emb):
    mesh = plsc.VectorSubcoreMesh(core_axis_name="c", subcore_axis_name="s",
                                  num_cores=NC, num_subcores=NS)
    return pl.kernel(
        _tec_body,
        out_type=jax.ShapeDtypeStruct((BATCH,), jnp.float32),
        mesh=mesh,
        compiler_params=pltpu.CompilerParams(needs_layout_passes=False,
                                             use_tc_tiling_on_sc=False),
        scratch_types=[
            pltpu.VMEM((N_CHUNKS, IDX_CHUNK), jnp.int32),
            pltpu.VMEM((N_CHUNKS, IDX_CHUNK), jnp.int32),
            pltpu.VMEM((N_CHUNKS, IDX_CHUNK), jnp.int32),
            pltpu.VMEM((B_PER_W, HIDDEN), jnp.float32),
            pltpu.VMEM((B_PER_W, HIDDEN), jnp.float32),
            pltpu.VMEM((B_PER_W, HIDDEN), jnp.float32),
            pltpu.VMEM((B_PER_W,), jnp.float32),
            pltpu.SemaphoreType.DMA,
        ],
    )(head, rel, tail, node_emb, rel_emb)


def kernel(head, rel, tail, node_emb, rel_emb):
    shp = (NW * N_CHUNKS, IDX_CHUNK)
    return _kge_score(head.astype(jnp.int32).reshape(shp),
                      rel.astype(jnp.int32).reshape(shp),
                      tail.astype(jnp.int32).reshape(shp),
                      node_emb, rel_emb)
